# baseline (speedup 1.0000x reference)
import functools

import jax
import jax.numpy as jnp
from jax import lax
from jax.experimental import pallas as pl
from jax.experimental.pallas import tpu as pltpu
from jax.experimental.pallas import tpu_sc as plsc

N_NODES = 10000
N_EDGES = 320000
D_IN = 128
D_OUT = 128

NC = 2
NS = 16
NW = NC * NS

CHUNK = 128
SEG = 8
EDGES_PER_WORKER = 10240
NCHUNKS = EDGES_PER_WORKER // CHUNK
NSEG = NCHUNKS // SEG
E_PAD = EDGES_PER_WORKER * NW

N_ACC = 10112
ROWS_PT = N_ACC // NS
DUMMY = N_NODES

N_DEG = 10240
DEG_ROWS = N_DEG // 16
DROWS_PT = DEG_ROWS // NS

BM = 2048


def _sc_aggregate(x, src2d, dst2d):
  mesh = plsc.VectorSubcoreMesh(core_axis_name="c", subcore_axis_name="s")

  @functools.partial(
      pl.kernel,
      out_type=(
          jax.ShapeDtypeStruct((NC, N_ACC, D_IN), jnp.float32),
          jax.ShapeDtypeStruct((NW * N_DEG,), jnp.float32),
      ),
      mesh=mesh,
      compiler_params=pltpu.CompilerParams(needs_layout_passes=False),
      scratch_types=[
          pltpu.VMEM((SEG, 1, CHUNK), jnp.int32),
          pltpu.VMEM((SEG, 1, CHUNK), jnp.int32),
          pltpu.VMEM((CHUNK, D_IN), jnp.float32),
          pltpu.VMEM((CHUNK, D_IN), jnp.float32),
          pltpu.VMEM((N_DEG,), jnp.float32),
          pltpu.VMEM_SHARED((N_ACC, D_IN), jnp.float32),
          pltpu.SemaphoreType.DMA,
          pltpu.SemaphoreType.DMA,
          pltpu.SemaphoreType.DMA,
          pltpu.SemaphoreType.DMA,
      ],
  )
  def agg(x_ref, src_ref, dst_ref, acc_out, deg_out, src_v, dst_v, rows0,
          rows1, hist, acc_sp, gsem0, gsem1, ssem0, ssem1):
    c = lax.axis_index("c")
    s = lax.axis_index("s")
    w = c * NS + s
    base = pl.multiple_of(s * ROWS_PT, 8)

    zeros = jnp.zeros((16,), jnp.float32)

    def z0(i, carry):
      rows0[i // 8, pl.ds((i % 8) * 16, 16)] = zeros
      return carry

    lax.fori_loop(0, CHUNK * 8, z0, 0)

    def z1(i, carry):
      hist[pl.ds(i * 16, 16)] = zeros
      return carry

    lax.fori_loop(0, DEG_ROWS, z1, 0)

    def zcopy(i, carry):
      pltpu.sync_copy(rows0, acc_sp.at[pl.ds(base + i * CHUNK, CHUNK)])
      return carry

    lax.fori_loop(0, ROWS_PT // CHUNK, zcopy, 0)
    rem = ROWS_PT % CHUNK
    if rem:
      pltpu.sync_copy(
          rows0.at[pl.ds(0, rem)],
          acc_sp.at[pl.ds(base + (ROWS_PT // CHUNK) * CHUNK, rem)])

    plsc.subcore_barrier()

    rows = (rows0, rows1)
    gsem = (gsem0, gsem1)
    ssem = (ssem0, ssem1)

    def gstart(j, b):
      pltpu.async_copy(x_ref.at[src_v.at[j, 0]], rows[b], gsem[b])

    def gwait(j, b):
      pltpu.make_async_copy(x_ref.at[src_v.at[j, 0]], rows[b], gsem[b]).wait()

    def sstart(j, b):
      pltpu.async_copy(rows[b], acc_sp.at[dst_v.at[j, 0]], ssem[b], add=True)

    def swait(j, b):
      pltpu.make_async_copy(rows[b], acc_sp.at[dst_v.at[j, 0]], ssem[b]).wait()

    ones16 = zeros + 1.0

    def hist_update(k):
      for u in range(8):
        dvec = dst_v[k, 0, pl.ds(u * 16, 16)]
        plsc.addupdate_scatter(hist, (dvec,), ones16)

    def seg_body(g, carry):
      off = pl.multiple_of(w * NCHUNKS + g * SEG, 8)
      pltpu.sync_copy(src_ref.at[pl.ds(off, SEG)], src_v)
      pltpu.sync_copy(dst_ref.at[pl.ds(off, SEG)], dst_v)
      gstart(0, 0)
      gstart(1, 1)
      for k in range(SEG - 2):
        b = k % 2
        gwait(k, b)
        sstart(k, b)
        hist_update(k)
        swait(k, b)
        gstart(k + 2, b)
      for k in range(SEG - 2, SEG):
        b = k % 2
        gwait(k, b)
        sstart(k, b)
        hist_update(k)
        swait(k, b)
      return carry

    lax.fori_loop(0, NSEG, seg_body, 0)

    pltpu.sync_copy(hist, deg_out.at[pl.ds(pl.multiple_of(w * N_DEG, 128), N_DEG)])

    plsc.subcore_barrier()

    pltpu.sync_copy(acc_sp.at[pl.ds(base, ROWS_PT)],
                    acc_out.at[c, pl.ds(base, ROWS_PT)])

  return agg(x, src2d, dst2d)


def _tc_body(x_ref, acc_ref, deg_ref, w_ref, m_ref, b_ref, o_ref):
  wm = w_ref[...] * m_ref[...]
  cs = acc_ref[0] + acc_ref[1]
  deg = jnp.reshape(jnp.sum(deg_ref[...], axis=0), (BM, 1))
  cmean = cs / jnp.maximum(deg, 1.0)
  h = (jnp.dot(x_ref[...], wm[:D_IN], preferred_element_type=jnp.float32)
       + jnp.dot(cmean, wm[D_IN:], preferred_element_type=jnp.float32)
       + b_ref[...])
  n = jnp.sqrt(jnp.sum(h * h, axis=1, keepdims=True))
  o_ref[...] = h / jnp.maximum(n, 1e-12)


def _tc_apply(x, acc, deg, W, mask, b2):
  return pl.pallas_call(
      _tc_body,
      grid=((N_NODES + BM - 1) // BM,),
      in_specs=[
          pl.BlockSpec((BM, D_IN), lambda i: (i, 0)),
          pl.BlockSpec((NC, BM, D_IN), lambda i: (0, i, 0)),
          pl.BlockSpec((NW, BM), lambda i: (0, i)),
          pl.BlockSpec((2 * D_IN, D_OUT), lambda i: (0, 0)),
          pl.BlockSpec((2 * D_IN, D_OUT), lambda i: (0, 0)),
          pl.BlockSpec((1, D_OUT), lambda i: (0, 0)),
      ],
      out_specs=pl.BlockSpec((BM, D_OUT), lambda i: (i, 0)),
      out_shape=jax.ShapeDtypeStruct((N_NODES, D_OUT), jnp.float32),
  )(x, acc, deg, W, mask, b2)


def kernel(x, edge_index, W, b, mask):
  x = x.astype(jnp.float32)
  ei = edge_index.astype(jnp.int32)
  npad = E_PAD - N_EDGES
  src = jnp.concatenate([ei[0], jnp.zeros((npad,), jnp.int32)])
  pad_dst = DUMMY + (jnp.arange(npad, dtype=jnp.int32) % (N_ACC - N_NODES))
  dst = jnp.concatenate([ei[1], pad_dst])
  src2d = src.reshape(NW * NCHUNKS, 1, CHUNK)
  dst2d = dst.reshape(NW * NCHUNKS, 1, CHUNK)
  acc, degp = _sc_aggregate(x, src2d, dst2d)
  deg = degp.reshape(NW, N_DEG)
  return _tc_apply(x, acc, deg, W, mask, b.reshape(1, D_OUT))


# speedup vs baseline: 1.0584x; 1.0584x over previous
import functools

import jax
import jax.numpy as jnp
from jax import lax
from jax.experimental import pallas as pl
from jax.experimental.pallas import tpu as pltpu
from jax.experimental.pallas import tpu_sc as plsc

N_NODES = 10000
N_EDGES = 320000
D_IN = 128
D_OUT = 128

NC = 2
NS = 16
NW = NC * NS

CHUNK = 128
SEG = 10
EDGES_PER_WORKER = 10240
NCHUNKS = EDGES_PER_WORKER // CHUNK
NSEG = NCHUNKS // SEG
E_PAD = EDGES_PER_WORKER * NW

N_ACC = 10112
ROWS_PT = N_ACC // NS
DUMMY = N_NODES

N_DEG = 10240
DEG_ROWS = N_DEG // 16
DROWS_PT = DEG_ROWS // NS

BM = 2048


def _sc_aggregate(x, src2d, dst2d):
  mesh = plsc.VectorSubcoreMesh(core_axis_name="c", subcore_axis_name="s")

  @functools.partial(
      pl.kernel,
      out_type=(
          jax.ShapeDtypeStruct((NC, N_ACC, D_IN), jnp.float32),
          jax.ShapeDtypeStruct((NW * N_DEG,), jnp.float32),
      ),
      mesh=mesh,
      compiler_params=pltpu.CompilerParams(
          use_tc_tiling_on_sc=False, needs_layout_passes=False),
      scratch_types=[
          pltpu.VMEM((SEG, CHUNK), jnp.int32),
          pltpu.VMEM((SEG, CHUNK), jnp.int32),
          pltpu.VMEM((CHUNK, D_IN), jnp.float32),
          pltpu.VMEM((CHUNK, D_IN), jnp.float32),
          pltpu.VMEM((N_DEG,), jnp.float32),
          pltpu.VMEM_SHARED((N_ACC, D_IN), jnp.float32),
          pltpu.SemaphoreType.DMA,
          pltpu.SemaphoreType.DMA,
          pltpu.SemaphoreType.DMA,
          pltpu.SemaphoreType.DMA,
      ],
  )
  def agg(x_ref, src_ref, dst_ref, acc_out, deg_out, src_v, dst_v, rows0,
          rows1, hist, acc_sp, gsem0, gsem1, ssem0, ssem1):
    c = lax.axis_index("c")
    s = lax.axis_index("s")
    w = c * NS + s
    base = pl.multiple_of(s * ROWS_PT, 8)

    zeros = jnp.zeros((16,), jnp.float32)

    def z0(i, carry):
      rows0[i // 8, pl.ds((i % 8) * 16, 16)] = zeros
      return carry

    lax.fori_loop(0, CHUNK * 8, z0, 0)

    def z1(i, carry):
      hist[pl.ds(i * 16, 16)] = zeros
      return carry

    lax.fori_loop(0, DEG_ROWS, z1, 0)

    def zcopy(i, carry):
      pltpu.sync_copy(rows0, acc_sp.at[pl.ds(base + i * CHUNK, CHUNK)])
      return carry

    lax.fori_loop(0, ROWS_PT // CHUNK, zcopy, 0)
    rem = ROWS_PT % CHUNK
    if rem:
      pltpu.sync_copy(
          rows0.at[pl.ds(0, rem)],
          acc_sp.at[pl.ds(base + (ROWS_PT // CHUNK) * CHUNK, rem)])

    plsc.subcore_barrier()

    rows = (rows0, rows1)
    gsem = (gsem0, gsem1)
    ssem = (ssem0, ssem1)

    def gstart(j, b):
      pltpu.async_copy(x_ref.at[src_v.at[j]], rows[b], gsem[b])

    def gwait(j, b):
      pltpu.make_async_copy(x_ref.at[src_v.at[j]], rows[b], gsem[b]).wait()

    def sstart(j, b):
      pltpu.async_copy(rows[b], acc_sp.at[dst_v.at[j]], ssem[b], add=True)

    def swait(j, b):
      pltpu.make_async_copy(rows[b], acc_sp.at[dst_v.at[j]], ssem[b]).wait()

    ones16 = zeros + 1.0

    def hist_update(k):
      for u in range(8):
        dvec = dst_v[k, pl.ds(u * 16, 16)]
        plsc.addupdate_scatter(hist, (dvec,), ones16)

    def seg_body(g, carry):
      off = pl.multiple_of(w * NCHUNKS + g * SEG, 8)
      pltpu.sync_copy(src_ref.at[pl.ds(off, SEG)], src_v)
      pltpu.sync_copy(dst_ref.at[pl.ds(off, SEG)], dst_v)
      gstart(0, 0)
      gstart(1, 1)
      for k in range(SEG - 2):
        b = k % 2
        gwait(k, b)
        sstart(k, b)
        hist_update(k)
        swait(k, b)
        gstart(k + 2, b)
      for k in range(SEG - 2, SEG):
        b = k % 2
        gwait(k, b)
        sstart(k, b)
        hist_update(k)
        swait(k, b)
      return carry

    lax.fori_loop(0, NSEG, seg_body, 0)

    pltpu.sync_copy(hist, deg_out.at[pl.ds(pl.multiple_of(w * N_DEG, 128), N_DEG)])

    plsc.subcore_barrier()

    pltpu.sync_copy(acc_sp.at[pl.ds(base, ROWS_PT)],
                    acc_out.at[c, pl.ds(base, ROWS_PT)])

  return agg(x, src2d, dst2d)


def _tc_body(x_ref, acc_ref, deg_ref, w_ref, m_ref, b_ref, o_ref):
  wm = w_ref[...] * m_ref[...]
  cs = acc_ref[0] + acc_ref[1]
  deg = jnp.reshape(jnp.sum(deg_ref[...], axis=0), (BM, 1))
  cmean = cs / jnp.maximum(deg, 1.0)
  h = (jnp.dot(x_ref[...], wm[:D_IN], preferred_element_type=jnp.float32)
       + jnp.dot(cmean, wm[D_IN:], preferred_element_type=jnp.float32)
       + b_ref[...])
  n = jnp.sqrt(jnp.sum(h * h, axis=1, keepdims=True))
  o_ref[...] = h / jnp.maximum(n, 1e-12)


def _tc_apply(x, acc, deg, W, mask, b2):
  return pl.pallas_call(
      _tc_body,
      grid=((N_NODES + BM - 1) // BM,),
      in_specs=[
          pl.BlockSpec((BM, D_IN), lambda i: (i, 0)),
          pl.BlockSpec((NC, BM, D_IN), lambda i: (0, i, 0)),
          pl.BlockSpec((NW, BM), lambda i: (0, i)),
          pl.BlockSpec((2 * D_IN, D_OUT), lambda i: (0, 0)),
          pl.BlockSpec((2 * D_IN, D_OUT), lambda i: (0, 0)),
          pl.BlockSpec((1, D_OUT), lambda i: (0, 0)),
      ],
      out_specs=pl.BlockSpec((BM, D_OUT), lambda i: (i, 0)),
      out_shape=jax.ShapeDtypeStruct((N_NODES, D_OUT), jnp.float32),
  )(x, acc, deg, W, mask, b2)


def kernel(x, edge_index, W, b, mask):
  x = x.astype(jnp.float32)
  ei = edge_index.astype(jnp.int32)
  npad = E_PAD - N_EDGES
  src = jnp.concatenate([ei[0], jnp.zeros((npad,), jnp.int32)])
  pad_dst = DUMMY + (jnp.arange(npad, dtype=jnp.int32) % (N_ACC - N_NODES))
  dst = jnp.concatenate([ei[1], pad_dst])
  src2d = src.reshape(NW * NCHUNKS, CHUNK)
  dst2d = dst.reshape(NW * NCHUNKS, CHUNK)
  acc, degp = _sc_aggregate(x, src2d, dst2d)
  deg = degp.reshape(NW, N_DEG)
  return _tc_apply(x, acc, deg, W, mask, b.reshape(1, D_OUT))


# speedup vs baseline: 1.1745x; 1.1097x over previous
import functools

import jax
import jax.numpy as jnp
from jax import lax
from jax.experimental import pallas as pl
from jax.experimental.pallas import tpu as pltpu
from jax.experimental.pallas import tpu_sc as plsc

N_NODES = 10000
N_EDGES = 320000
D_IN = 128
D_OUT = 128

D_AUG = 144
DEG_COL = 128

NC = 2
NS = 16
NW = NC * NS

CHUNK = 128
EDGES_PER_WORKER = 10240
NCHUNKS = EDGES_PER_WORKER // CHUNK
SEG = 10
NSEG = NCHUNKS // SEG
E_PAD = EDGES_PER_WORKER * NW

N_ACC = 10112
ROWS_PT = N_ACC // NS
DUMMY = N_NODES

BM = 2000


def _sc_aggregate(x_aug, src2d, dst2d):
  mesh = plsc.VectorSubcoreMesh(core_axis_name="c", subcore_axis_name="s")

  @functools.partial(
      pl.kernel,
      out_type=jax.ShapeDtypeStruct((NC, N_ACC, D_AUG), jnp.float32),
      mesh=mesh,
      compiler_params=pltpu.CompilerParams(use_tc_tiling_on_sc=False),
      scratch_types=[
          pltpu.VMEM((SEG, CHUNK), jnp.int32),
          pltpu.VMEM((SEG, CHUNK), jnp.int32),
          pltpu.VMEM((CHUNK, D_AUG), jnp.float32),
          pltpu.VMEM((CHUNK, D_AUG), jnp.float32),
          pltpu.VMEM_SHARED((N_ACC, D_AUG), jnp.float32),
          pltpu.SemaphoreType.DMA,
          pltpu.SemaphoreType.DMA,
          pltpu.SemaphoreType.DMA,
          pltpu.SemaphoreType.DMA,
          pltpu.SemaphoreType.DMA,
      ],
  )
  def agg(x_ref, src_ref, dst_ref, out_ref, src_v, dst_v, rows0, rows1,
          acc_sp, gsem0, gsem1, ssem0, ssem1, isem):
    c = lax.axis_index("c")
    s = lax.axis_index("s")
    w = c * NS + s
    base = s * ROWS_PT
    rows = (rows0, rows1)
    gsem = (gsem0, gsem1)
    ssem = (ssem0, ssem1)

    zeros = jnp.zeros((16,), jnp.float32)
    nseg = D_AUG // 16

    def zbody(i, carry):
      rows0[i // nseg, pl.ds((i % nseg) * 16, 16)] = zeros
      return carry

    lax.fori_loop(0, CHUNK * nseg, zbody, 0)

    def zcopy(i, carry):
      pltpu.sync_copy(rows0, acc_sp.at[pl.ds(base + i * CHUNK, CHUNK)])
      return carry

    lax.fori_loop(0, ROWS_PT // CHUNK, zcopy, 0)
    rem = ROWS_PT % CHUNK
    if rem:
      pltpu.sync_copy(
          rows0.at[pl.ds(0, rem)],
          acc_sp.at[pl.ds(base + (ROWS_PT // CHUNK) * CHUNK, rem)])

    plsc.subcore_barrier()

    rows = (rows0, rows1)
    gsem = (gsem0, gsem1)
    ssem = (ssem0, ssem1)

    def gstart(j, b):
      pltpu.async_copy(x_ref.at[src_v.at[j]], rows[b], gsem[b])

    def gwait(j, b):
      pltpu.make_async_copy(x_ref.at[src_v.at[j]], rows[b], gsem[b]).wait()

    def sstart(j, b):
      pltpu.async_copy(rows[b], acc_sp.at[dst_v.at[j]], ssem[b], add=True)

    def swait(j, b):
      pltpu.make_async_copy(rows[b], acc_sp.at[dst_v.at[j]], ssem[b]).wait()

    def seg_body(g, carry):
      off = w * NCHUNKS + g * SEG
      pltpu.sync_copy(src_ref.at[pl.ds(off, SEG)], src_v)
      pltpu.sync_copy(dst_ref.at[pl.ds(off, SEG)], dst_v)
      gstart(0, 0)
      gstart(1, 1)
      for k in range(SEG - 2):
        b = k % 2
        gwait(k, b)
        sstart(k, b)
        swait(k, b)
        gstart(k + 2, b)
      for k in range(SEG - 2, SEG):
        b = k % 2
        gwait(k, b)
        sstart(k, b)
        swait(k, b)
      return carry

    lax.fori_loop(0, NSEG, seg_body, 0)

    plsc.subcore_barrier()

    pltpu.sync_copy(acc_sp.at[pl.ds(base, ROWS_PT)],
                    out_ref.at[c, pl.ds(base, ROWS_PT)])

  return agg(x_aug, src2d, dst2d)


def _tc_body(x_ref, acc_ref, w_ref, m_ref, b_ref, o_ref):
  wm = w_ref[...] * m_ref[...]
  cs = acc_ref[0] + acc_ref[1]
  deg = cs[:, DEG_COL:DEG_COL + 1]
  cmean = cs[:, :D_IN] / jnp.maximum(deg, 1.0)
  h = (jnp.dot(x_ref[...], wm[:D_IN], preferred_element_type=jnp.float32)
       + jnp.dot(cmean, wm[D_IN:], preferred_element_type=jnp.float32)
       + b_ref[...])
  n = jnp.sqrt(jnp.sum(h * h, axis=1, keepdims=True))
  o_ref[...] = h / jnp.maximum(n, 1e-12)


def _tc_apply(x, acc, W, mask, b2):
  return pl.pallas_call(
      _tc_body,
      grid=(N_NODES // BM,),
      in_specs=[
          pl.BlockSpec((BM, D_IN), lambda i: (i, 0)),
          pl.BlockSpec((NC, BM, D_AUG), lambda i: (0, i, 0)),
          pl.BlockSpec((2 * D_IN, D_OUT), lambda i: (0, 0)),
          pl.BlockSpec((2 * D_IN, D_OUT), lambda i: (0, 0)),
          pl.BlockSpec((1, D_OUT), lambda i: (0, 0)),
      ],
      out_specs=pl.BlockSpec((BM, D_OUT), lambda i: (i, 0)),
      out_shape=jax.ShapeDtypeStruct((N_NODES, D_OUT), jnp.float32),
  )(x, acc, W, mask, b2)


def kernel(x, edge_index, W, b, mask):
  x = x.astype(jnp.float32)
  ei = edge_index.astype(jnp.int32)
  npad = E_PAD - N_EDGES
  src = jnp.concatenate([ei[0], jnp.zeros((npad,), jnp.int32)])
  pad_dst = DUMMY + (jnp.arange(npad, dtype=jnp.int32) % (N_ACC - N_NODES))
  dst = jnp.concatenate([ei[1], pad_dst])
  src2d = src.reshape(NW * NCHUNKS, CHUNK)
  dst2d = dst.reshape(NW * NCHUNKS, CHUNK)
  x_aug = jnp.concatenate(
      [x, jnp.ones((N_NODES, 1), jnp.float32),
       jnp.zeros((N_NODES, D_AUG - D_IN - 1), jnp.float32)], axis=1)
  acc = _sc_aggregate(x_aug, src2d, dst2d)
  return _tc_apply(x, acc, W, mask, b.reshape(1, D_OUT))
